# Initial kernel scaffold; baseline (speedup 1.0000x reference)
import jax
import jax.numpy as jnp
from jax.experimental import pallas as pl


def kernel(x, edge_index, batch, W1, b1, g1, be1, W2, b2, g2, be2, W3, b3, g3, be3, Wc1, bc1, Wc2, bc2):
    raise NotImplementedError("write your pallas kernel here")


# speedup vs baseline: 11.6669x; 11.6669x over previous
import functools

import jax
import jax.numpy as jnp
from jax import lax
from jax.experimental import pallas as pl
from jax.experimental.pallas import tpu as pltpu
from jax.experimental.pallas import tpu_sc as plsc

N = 10000
E = 320000
D = 128
H = 64
O = 32
G = 64
EPS = 1e-5

NC = 2
NS = 16
NW = NC * NS
CHUNK = 128
NPAD = 10240
DEGW = 16

NCPT = (-(-E // (CHUNK * NW)) + 7) // 8 * 8
EPAD = NCPT * NW * CHUNK


def _zero_rows(ref, nrows, width):
  zero = jnp.zeros((16,), jnp.float32)
  per_row = width // 16

  def body(i, _):
    r = i // per_row
    k = i % per_row
    ref[r, pl.ds(k * 16, 16)] = zero
    return 0

  lax.fori_loop(0, nrows * per_row, body, 0)


def _fill_rows(ref, nrows, width, value):
  vec = jnp.full((16,), value, jnp.float32)
  per_row = width // 16

  def body(i, _):
    r = i // per_row
    k = i % per_row
    ref[r, pl.ds(k * 16, 16)] = vec
    return 0

  lax.fori_loop(0, nrows * per_row, body, 0)


def _make_msgpass(width):
  rows_per_tile = NPAD // NS
  drain_steps = rows_per_tile // CHUNK
  mesh = plsc.VectorSubcoreMesh(
      core_axis_name="c", subcore_axis_name="s",
      num_cores=NC, num_subcores=NS)

  @functools.partial(
      pl.kernel,
      out_type=jax.ShapeDtypeStruct((NC, NPAD, width), jnp.float32),
      mesh=mesh,
      compiler_params=pltpu.CompilerParams(use_tc_tiling_on_sc=False),
      scratch_types=[
          pltpu.VMEM((NCPT, CHUNK), jnp.int32),
          pltpu.VMEM((NCPT, CHUNK), jnp.int32),
          pltpu.VMEM((CHUNK, width), jnp.float32),
          pltpu.VMEM_SHARED((NPAD, width), jnp.float32),
          pltpu.SemaphoreType.DMA,
      ],
  )
  def msgpass(src_hbm, dst_hbm, feat_hbm, out_hbm,
              src_v, dst_v, rows_v, agg_sh, sem):
    c = lax.axis_index("c")
    s = lax.axis_index("s")
    wid = c * NS + s

    _zero_rows(rows_v, CHUNK, width)

    def zbody(i, _):
      pltpu.sync_copy(rows_v, agg_sh.at[pl.ds(s * rows_per_tile + i * CHUNK,
                                              CHUNK)])
      return 0

    lax.fori_loop(0, drain_steps, zbody, 0)

    pltpu.sync_copy(src_hbm.at[wid], src_v)
    pltpu.sync_copy(dst_hbm.at[wid], dst_v)

    plsc.subcore_barrier()

    def chunk_body(j, _):
      pltpu.async_copy(feat_hbm.at[src_v.at[j]], rows_v, sem).wait()
      pltpu.sync_copy(rows_v, agg_sh.at[dst_v.at[j]], add=True)
      return 0

    lax.fori_loop(0, NCPT, chunk_body, 0)

    plsc.subcore_barrier()

    def drain_body(i, _):
      r0 = s * rows_per_tile + i * CHUNK
      pltpu.sync_copy(agg_sh.at[pl.ds(r0, CHUNK)], rows_v)
      pltpu.sync_copy(rows_v, out_hbm.at[c, pl.ds(r0, CHUNK)])
      return 0

    lax.fori_loop(0, drain_steps, drain_body, 0)

  return msgpass


def _make_degree():
  rows_per_tile = NPAD // NS
  drain_steps = rows_per_tile // CHUNK
  mesh = plsc.VectorSubcoreMesh(
      core_axis_name="c", subcore_axis_name="s",
      num_cores=NC, num_subcores=NS)

  @functools.partial(
      pl.kernel,
      out_type=jax.ShapeDtypeStruct((NC, NPAD, DEGW), jnp.float32),
      mesh=mesh,
      compiler_params=pltpu.CompilerParams(use_tc_tiling_on_sc=False),
      scratch_types=[
          pltpu.VMEM((NCPT, CHUNK), jnp.int32),
          pltpu.VMEM((CHUNK, DEGW), jnp.float32),
          pltpu.VMEM_SHARED((NPAD, DEGW), jnp.float32),
      ],
  )
  def degree(dst_hbm, out_hbm, dst_v, buf_v, deg_sh):
    c = lax.axis_index("c")
    s = lax.axis_index("s")
    wid = c * NS + s

    _zero_rows(buf_v, CHUNK, DEGW)

    def zbody(i, _):
      pltpu.sync_copy(buf_v, deg_sh.at[pl.ds(s * rows_per_tile + i * CHUNK,
                                             CHUNK)])
      return 0

    lax.fori_loop(0, drain_steps, zbody, 0)

    pltpu.sync_copy(dst_hbm.at[wid], dst_v)
    _fill_rows(buf_v, CHUNK, DEGW, 1.0)

    plsc.subcore_barrier()

    def chunk_body(j, _):
      pltpu.sync_copy(buf_v, deg_sh.at[dst_v.at[j]], add=True)
      return 0

    lax.fori_loop(0, NCPT, chunk_body, 0)

    plsc.subcore_barrier()

    def drain_body(i, _):
      r0 = s * rows_per_tile + i * CHUNK
      pltpu.sync_copy(deg_sh.at[pl.ds(r0, CHUNK)], buf_v)
      pltpu.sync_copy(buf_v, out_hbm.at[c, pl.ds(r0, CHUNK)])
      return 0

    lax.fori_loop(0, drain_steps, drain_body, 0)

  return degree


def _dot_t(a, w):
  return lax.dot_general(a, w, (((1,), (1,)), ((), ())),
                         preferred_element_type=jnp.float32)


def _tc_first(degp, x, w1):
  def body(degp_ref, x_ref, w1_ref, dis_ref, hs1_ref):
    deg = (degp_ref[0, :N, 0:1] + degp_ref[1, :N, 0:1]) + 1.0
    dis = lax.rsqrt(deg)
    h1 = _dot_t(x_ref[...], w1_ref[...])
    dis_ref[...] = dis
    hs1_ref[...] = h1 * dis

  return pl.pallas_call(
      body,
      out_shape=[
          jax.ShapeDtypeStruct((N, 1), jnp.float32),
          jax.ShapeDtypeStruct((N, H), jnp.float32),
      ],
  )(degp, x, w1)


def _tc_mid(aggp, hs, dis, b, gam, bet, wn, width):
  wn_out = wn.shape[0]

  def body(aggp_ref, hs_ref, dis_ref, b_ref, g_ref, be_ref, wn_ref, o_ref):
    dis = dis_ref[...]
    agg = aggp_ref[0, :N, :] + aggp_ref[1, :N, :] + hs_ref[...]
    out = agg * dis + b_ref[...]
    mu = jnp.mean(out, axis=0, keepdims=True)
    xc = out - mu
    var = jnp.mean(xc * xc, axis=0, keepdims=True)
    y = g_ref[...] * xc * lax.rsqrt(var + EPS) + be_ref[...]
    y = jnp.maximum(y, 0.0)
    o_ref[...] = _dot_t(y, wn_ref[...]) * dis

  return pl.pallas_call(
      body,
      out_shape=jax.ShapeDtypeStruct((N, wn_out), jnp.float32),
  )(aggp, hs, dis, b.reshape(1, width), gam.reshape(1, width),
    bet.reshape(1, width), wn)


def _tc_final(aggp, hs, dis, b, gam, bet, batch2d, wc1, bc1, wc2, bc2):
  def body(aggp_ref, hs_ref, dis_ref, b_ref, g_ref, be_ref, batch_ref,
           wc1_ref, bc1_ref, wc2_ref, bc2_ref, logit_ref, gcat_ref):
    dis = dis_ref[...]
    agg = aggp_ref[0, :N, :] + aggp_ref[1, :N, :] + hs_ref[...]
    out = agg * dis + b_ref[...]
    mu = jnp.mean(out, axis=0, keepdims=True)
    xc = out - mu
    var = jnp.mean(xc * xc, axis=0, keepdims=True)
    h = g_ref[...] * xc * lax.rsqrt(var + EPS) + be_ref[...]
    h = jnp.maximum(h, 0.0)

    batch = batch_ref[...]
    gids = lax.broadcasted_iota(jnp.int32, (1, G), 1)
    onehot = (batch == gids).astype(jnp.float32)
    h_aug = jnp.concatenate([h, jnp.ones((N, 1), jnp.float32)], axis=1)
    sums_aug = lax.dot_general(onehot, h_aug, (((0,), (0,)), ((), ())),
                               preferred_element_type=jnp.float32)
    counts = sums_aug[:, O:O + 1]
    mean_pool = sums_aug[:, :O] / jnp.maximum(counts, 1.0)
    gcat_ref[:, :O] = mean_pool

    def max_body(gi, _):
      mask = batch == gi
      m = jnp.max(jnp.where(mask, h, -jnp.inf), axis=0, keepdims=True)
      gcat_ref[pl.ds(gi, 1), pl.ds(O, O)] = m
      return 0

    lax.fori_loop(0, G, max_body, 0)

    gcat = gcat_ref[...]
    z = jnp.maximum(_dot_t(gcat, wc1_ref[...]) + bc1_ref[...], 0.0)
    logit_ref[...] = jnp.sum(z * wc2_ref[...], axis=1, keepdims=True) + bc2_ref[0, 0]

  return pl.pallas_call(
      body,
      out_shape=[
          jax.ShapeDtypeStruct((G, 1), jnp.float32),
          jax.ShapeDtypeStruct((G, 2 * O), jnp.float32),
      ],
  )(aggp, hs, dis, b.reshape(1, O), gam.reshape(1, O), bet.reshape(1, O),
    batch2d, wc1, bc1.reshape(1, O), wc2, bc2.reshape(1, 1))


_msgpass_h = _make_msgpass(H)
_msgpass_o = _make_msgpass(O)
_degree = _make_degree()


def kernel(x, edge_index, batch,
           W1, b1, g1, be1,
           W2, b2, g2, be2,
           W3, b3, g3, be3,
           Wc1, bc1, Wc2, bc2):
  pad = EPAD - E
  src = jnp.concatenate([edge_index[0], jnp.zeros((pad,), jnp.int32)])
  dst = jnp.concatenate([edge_index[1], jnp.full((pad,), N, jnp.int32)])
  src_m = src.reshape(NW, NCPT, CHUNK)
  dst_m = dst.reshape(NW, NCPT, CHUNK)
  batch2d = batch.reshape(N, 1)

  degp = _degree(dst_m)
  dis, hs1 = _tc_first(degp, x, W1)
  agg1 = _msgpass_h(src_m, dst_m, hs1)
  hs2 = _tc_mid(agg1, hs1, dis, b1, g1, be1, W2, H)
  agg2 = _msgpass_h(src_m, dst_m, hs2)
  hs3 = _tc_mid(agg2, hs2, dis, b2, g2, be2, W3, H)
  agg3 = _msgpass_o(src_m, dst_m, hs3)
  logit2d, gcat = _tc_final(agg3, hs3, dis, b3, g3, be3, batch2d,
                            Wc1, bc1, Wc2, bc2)
  return logit2d.reshape(-1), gcat


# speedup vs baseline: 13.4016x; 1.1487x over previous
import functools

import jax
import jax.numpy as jnp
from jax import lax
from jax.experimental import pallas as pl
from jax.experimental.pallas import tpu as pltpu
from jax.experimental.pallas import tpu_sc as plsc

N = 10000
E = 320000
D = 128
H = 64
O = 32
G = 64
EPS = 1e-5

NC = 2
NS = 16
NW = NC * NS
CHUNK = 128
NPAD = 10240
DEGW = 16

NCPT = (-(-E // (CHUNK * NW)) + 7) // 8 * 8
EPAD = NCPT * NW * CHUNK


def _zero_rows(ref, nrows, width):
  zero = jnp.zeros((16,), jnp.float32)
  per_row = width // 16

  def body(i, _):
    r = i // per_row
    k = i % per_row
    ref[r, pl.ds(k * 16, 16)] = zero
    return 0

  lax.fori_loop(0, nrows * per_row, body, 0)


def _fill_rows(ref, nrows, width, value):
  vec = jnp.full((16,), value, jnp.float32)
  per_row = width // 16

  def body(i, _):
    r = i // per_row
    k = i % per_row
    ref[r, pl.ds(k * 16, 16)] = vec
    return 0

  lax.fori_loop(0, nrows * per_row, body, 0)


NBUF = 8


def _make_msgpass(width):
  rows_per_tile = NPAD // NS
  drain_steps = rows_per_tile // CHUNK
  nrounds = NCPT // NBUF
  mesh = plsc.VectorSubcoreMesh(
      core_axis_name="c", subcore_axis_name="s",
      num_cores=NC, num_subcores=NS)

  @functools.partial(
      pl.kernel,
      out_type=jax.ShapeDtypeStruct((NC, NPAD, width), jnp.float32),
      mesh=mesh,
      compiler_params=pltpu.CompilerParams(use_tc_tiling_on_sc=False),
      scratch_types=[
          pltpu.VMEM((NCPT, CHUNK), jnp.int32),
          pltpu.VMEM((NCPT, CHUNK), jnp.int32),
          pltpu.VMEM((NBUF, CHUNK, width), jnp.float32),
          pltpu.VMEM_SHARED((NPAD, width), jnp.float32),
          [pltpu.SemaphoreType.DMA] * NBUF,
          [pltpu.SemaphoreType.DMA] * NBUF,
      ],
  )
  def msgpass(src_hbm, dst_hbm, feat_hbm, out_hbm,
              src_v, dst_v, rows_v, agg_sh, gsem, ssem):
    c = lax.axis_index("c")
    s = lax.axis_index("s")
    wid = c * NS + s

    _zero_rows(rows_v.at[0], CHUNK, width)

    def zbody(i, _):
      pltpu.sync_copy(rows_v.at[0],
                      agg_sh.at[pl.ds(s * rows_per_tile + i * CHUNK, CHUNK)])
      return 0

    lax.fori_loop(0, drain_steps, zbody, 0)

    pltpu.sync_copy(src_hbm.at[wid], src_v)
    pltpu.sync_copy(dst_hbm.at[wid], dst_v)

    plsc.subcore_barrier()

    for b in range(NBUF):
      pltpu.async_copy(feat_hbm.at[src_v.at[b]], rows_v.at[b], gsem[b])

    def round_body(t, _):
      for b in range(NBUF):
        j = t * NBUF + b
        pltpu.make_async_copy(feat_hbm.at[src_v.at[0]], rows_v.at[b],
                              gsem[b]).wait()
        pltpu.async_copy(rows_v.at[b], agg_sh.at[dst_v.at[j]], ssem[b],
                         add=True)
      for b in range(NBUF):
        jn = t * NBUF + b + NBUF

        @pl.when(jn < NCPT)
        def _():
          pltpu.make_async_copy(rows_v.at[b], agg_sh.at[dst_v.at[0]],
                                ssem[b]).wait()
          pltpu.async_copy(feat_hbm.at[src_v.at[jn]], rows_v.at[b], gsem[b])

      return 0

    lax.fori_loop(0, nrounds, round_body, 0)
    for b in range(NBUF):
      pltpu.make_async_copy(rows_v.at[b], agg_sh.at[dst_v.at[0]],
                            ssem[b]).wait()

    plsc.subcore_barrier()

    def drain_body(i, _):
      r0 = s * rows_per_tile + i * CHUNK
      pltpu.sync_copy(agg_sh.at[pl.ds(r0, CHUNK)], rows_v.at[0])
      pltpu.sync_copy(rows_v.at[0], out_hbm.at[c, pl.ds(r0, CHUNK)])
      return 0

    lax.fori_loop(0, drain_steps, drain_body, 0)

  return msgpass


def _make_degree():
  rows_per_tile = NPAD // NS
  drain_steps = rows_per_tile // CHUNK
  mesh = plsc.VectorSubcoreMesh(
      core_axis_name="c", subcore_axis_name="s",
      num_cores=NC, num_subcores=NS)

  @functools.partial(
      pl.kernel,
      out_type=jax.ShapeDtypeStruct((NC, NPAD, DEGW), jnp.float32),
      mesh=mesh,
      compiler_params=pltpu.CompilerParams(use_tc_tiling_on_sc=False),
      scratch_types=[
          pltpu.VMEM((NCPT, CHUNK), jnp.int32),
          pltpu.VMEM((CHUNK, DEGW), jnp.float32),
          pltpu.VMEM_SHARED((NPAD, DEGW), jnp.float32),
          [pltpu.SemaphoreType.DMA] * NBUF,
      ],
  )
  def degree(dst_hbm, out_hbm, dst_v, buf_v, deg_sh, ssem):
    c = lax.axis_index("c")
    s = lax.axis_index("s")
    wid = c * NS + s

    _zero_rows(buf_v, CHUNK, DEGW)

    def zbody(i, _):
      pltpu.sync_copy(buf_v, deg_sh.at[pl.ds(s * rows_per_tile + i * CHUNK,
                                             CHUNK)])
      return 0

    lax.fori_loop(0, drain_steps, zbody, 0)

    pltpu.sync_copy(dst_hbm.at[wid], dst_v)
    _fill_rows(buf_v, CHUNK, DEGW, 1.0)

    plsc.subcore_barrier()

    nrounds = NCPT // NBUF
    for b in range(NBUF):
      pltpu.async_copy(buf_v, deg_sh.at[dst_v.at[b]], ssem[b], add=True)

    def round_body(t, _):
      for b in range(NBUF):
        jn = t * NBUF + b + NBUF

        @pl.when(jn < NCPT)
        def _():
          pltpu.make_async_copy(buf_v, deg_sh.at[dst_v.at[0]], ssem[b]).wait()
          pltpu.async_copy(buf_v, deg_sh.at[dst_v.at[jn]], ssem[b], add=True)

      return 0

    lax.fori_loop(0, nrounds, round_body, 0)
    for b in range(NBUF):
      pltpu.make_async_copy(buf_v, deg_sh.at[dst_v.at[0]], ssem[b]).wait()

    plsc.subcore_barrier()

    def drain_body(i, _):
      r0 = s * rows_per_tile + i * CHUNK
      pltpu.sync_copy(deg_sh.at[pl.ds(r0, CHUNK)], buf_v)
      pltpu.sync_copy(buf_v, out_hbm.at[c, pl.ds(r0, CHUNK)])
      return 0

    lax.fori_loop(0, drain_steps, drain_body, 0)

  return degree


def _dot_t(a, w):
  return lax.dot_general(a, w, (((1,), (1,)), ((), ())),
                         preferred_element_type=jnp.float32)


def _tc_first(degp, x, w1):
  def body(degp_ref, x_ref, w1_ref, dis_ref, hs1_ref):
    deg = (degp_ref[0, :N, 0:1] + degp_ref[1, :N, 0:1]) + 1.0
    dis = lax.rsqrt(deg)
    h1 = _dot_t(x_ref[...], w1_ref[...])
    dis_ref[...] = dis
    hs1_ref[...] = h1 * dis

  return pl.pallas_call(
      body,
      out_shape=[
          jax.ShapeDtypeStruct((N, 1), jnp.float32),
          jax.ShapeDtypeStruct((N, H), jnp.float32),
      ],
  )(degp, x, w1)


def _tc_mid(aggp, hs, dis, b, gam, bet, wn, width):
  wn_out = wn.shape[0]

  def body(aggp_ref, hs_ref, dis_ref, b_ref, g_ref, be_ref, wn_ref, o_ref):
    dis = dis_ref[...]
    agg = aggp_ref[0, :N, :] + aggp_ref[1, :N, :] + hs_ref[...]
    out = agg * dis + b_ref[...]
    mu = jnp.mean(out, axis=0, keepdims=True)
    xc = out - mu
    var = jnp.mean(xc * xc, axis=0, keepdims=True)
    y = g_ref[...] * xc * lax.rsqrt(var + EPS) + be_ref[...]
    y = jnp.maximum(y, 0.0)
    o_ref[...] = _dot_t(y, wn_ref[...]) * dis

  return pl.pallas_call(
      body,
      out_shape=jax.ShapeDtypeStruct((N, wn_out), jnp.float32),
  )(aggp, hs, dis, b.reshape(1, width), gam.reshape(1, width),
    bet.reshape(1, width), wn)


def _tc_final(aggp, hs, dis, b, gam, bet, batch2d, wc1, bc1, wc2, bc2):
  def body(aggp_ref, hs_ref, dis_ref, b_ref, g_ref, be_ref, batch_ref,
           wc1_ref, bc1_ref, wc2_ref, bc2_ref, logit_ref, gcat_ref):
    dis = dis_ref[...]
    agg = aggp_ref[0, :N, :] + aggp_ref[1, :N, :] + hs_ref[...]
    out = agg * dis + b_ref[...]
    mu = jnp.mean(out, axis=0, keepdims=True)
    xc = out - mu
    var = jnp.mean(xc * xc, axis=0, keepdims=True)
    h = g_ref[...] * xc * lax.rsqrt(var + EPS) + be_ref[...]
    h = jnp.maximum(h, 0.0)

    batch = batch_ref[...]
    gids = lax.broadcasted_iota(jnp.int32, (1, G), 1)
    onehot = (batch == gids).astype(jnp.float32)
    h_aug = jnp.concatenate([h, jnp.ones((N, 1), jnp.float32)], axis=1)
    sums_aug = lax.dot_general(onehot, h_aug, (((0,), (0,)), ((), ())),
                               preferred_element_type=jnp.float32)
    counts = sums_aug[:, O:O + 1]
    mean_pool = sums_aug[:, :O] / jnp.maximum(counts, 1.0)
    gcat_ref[:, :O] = mean_pool

    def max_body(gi, _):
      mask = batch == gi
      m = jnp.max(jnp.where(mask, h, -jnp.inf), axis=0, keepdims=True)
      gcat_ref[pl.ds(gi, 1), pl.ds(O, O)] = m
      return 0

    lax.fori_loop(0, G, max_body, 0)

    gcat = gcat_ref[...]
    z = jnp.maximum(_dot_t(gcat, wc1_ref[...]) + bc1_ref[...], 0.0)
    logit_ref[...] = jnp.sum(z * wc2_ref[...], axis=1, keepdims=True) + bc2_ref[0, 0]

  return pl.pallas_call(
      body,
      out_shape=[
          jax.ShapeDtypeStruct((G, 1), jnp.float32),
          jax.ShapeDtypeStruct((G, 2 * O), jnp.float32),
      ],
  )(aggp, hs, dis, b.reshape(1, O), gam.reshape(1, O), bet.reshape(1, O),
    batch2d, wc1, bc1.reshape(1, O), wc2, bc2.reshape(1, 1))


_msgpass_h = _make_msgpass(H)
_msgpass_o = _make_msgpass(O)
_degree = _make_degree()


def kernel(x, edge_index, batch,
           W1, b1, g1, be1,
           W2, b2, g2, be2,
           W3, b3, g3, be3,
           Wc1, bc1, Wc2, bc2):
  pad = EPAD - E
  src = jnp.concatenate([edge_index[0], jnp.zeros((pad,), jnp.int32)])
  dst = jnp.concatenate([edge_index[1], jnp.full((pad,), N, jnp.int32)])
  src_m = src.reshape(NW, NCPT, CHUNK)
  dst_m = dst.reshape(NW, NCPT, CHUNK)
  batch2d = batch.reshape(N, 1)

  degp = _degree(dst_m)
  dis, hs1 = _tc_first(degp, x, W1)
  agg1 = _msgpass_h(src_m, dst_m, hs1)
  hs2 = _tc_mid(agg1, hs1, dis, b1, g1, be1, W2, H)
  agg2 = _msgpass_h(src_m, dst_m, hs2)
  hs3 = _tc_mid(agg2, hs2, dis, b2, g2, be2, W3, H)
  agg3 = _msgpass_o(src_m, dst_m, hs3)
  logit2d, gcat = _tc_final(agg3, hs3, dis, b3, g3, be3, batch2d,
                            Wc1, bc1, Wc2, bc2)
  return logit2d.reshape(-1), gcat


# speedup vs baseline: 13.5031x; 1.0076x over previous
import functools

import jax
import jax.numpy as jnp
from jax import lax
from jax.experimental import pallas as pl
from jax.experimental.pallas import tpu as pltpu
from jax.experimental.pallas import tpu_sc as plsc

N = 10000
E = 320000
D = 128
H = 64
O = 32
G = 64
EPS = 1e-5

NC = 2
NS = 16
NW = NC * NS
CHUNK = 128
NPAD = 10240
DEGW = 16

NCPT = (-(-E // (CHUNK * NW)) + 7) // 8 * 8
EPAD = NCPT * NW * CHUNK


def _zero_rows(ref, nrows, width):
  zero = jnp.zeros((16,), jnp.float32)
  per_row = width // 16

  def body(i, _):
    r = i // per_row
    k = i % per_row
    ref[r, pl.ds(k * 16, 16)] = zero
    return 0

  lax.fori_loop(0, nrows * per_row, body, 0)


def _fill_rows(ref, nrows, width, value):
  vec = jnp.full((16,), value, jnp.float32)
  per_row = width // 16

  def body(i, _):
    r = i // per_row
    k = i % per_row
    ref[r, pl.ds(k * 16, 16)] = vec
    return 0

  lax.fori_loop(0, nrows * per_row, body, 0)


NBUF = 8


def _make_msgpass(width):
  rows_per_tile = NPAD // NS
  drain_steps = rows_per_tile // CHUNK
  nrounds = NCPT // NBUF
  mesh = plsc.VectorSubcoreMesh(
      core_axis_name="c", subcore_axis_name="s",
      num_cores=NC, num_subcores=NS)

  @functools.partial(
      pl.kernel,
      out_type=jax.ShapeDtypeStruct((NC, NPAD, width), jnp.float32),
      mesh=mesh,
      compiler_params=pltpu.CompilerParams(use_tc_tiling_on_sc=False),
      scratch_types=[
          pltpu.VMEM((NCPT, CHUNK), jnp.int32),
          pltpu.VMEM((NCPT, CHUNK), jnp.int32),
          pltpu.VMEM((NBUF, CHUNK, width), jnp.float32),
          pltpu.VMEM_SHARED((NPAD, width), jnp.float32),
          [pltpu.SemaphoreType.DMA] * NBUF,
          [pltpu.SemaphoreType.DMA] * NBUF,
      ],
  )
  def msgpass(src_hbm, dst_hbm, feat_hbm, out_hbm,
              src_v, dst_v, rows_v, agg_sh, gsem, ssem):
    c = lax.axis_index("c")
    s = lax.axis_index("s")
    wid = c * NS + s

    _zero_rows(rows_v.at[0], CHUNK, width)

    def zbody(i, _):
      pltpu.sync_copy(rows_v.at[0],
                      agg_sh.at[pl.ds(s * rows_per_tile + i * CHUNK, CHUNK)])
      return 0

    lax.fori_loop(0, drain_steps, zbody, 0)

    pltpu.sync_copy(src_hbm.at[wid], src_v)
    pltpu.sync_copy(dst_hbm.at[wid], dst_v)

    plsc.subcore_barrier()

    for b in range(NBUF):
      pltpu.async_copy(feat_hbm.at[src_v.at[b]], rows_v.at[b], gsem[b])

    def round_body(t, _):
      for b in range(NBUF):
        j = t * NBUF + b
        pltpu.make_async_copy(feat_hbm.at[src_v.at[0]], rows_v.at[b],
                              gsem[b]).wait()
        pltpu.async_copy(rows_v.at[b], agg_sh.at[dst_v.at[j]], ssem[b],
                         add=True)
      for b in range(NBUF):
        jn = t * NBUF + b + NBUF

        @pl.when(jn < NCPT)
        def _():
          pltpu.make_async_copy(rows_v.at[b], agg_sh.at[dst_v.at[0]],
                                ssem[b]).wait()
          pltpu.async_copy(feat_hbm.at[src_v.at[jn]], rows_v.at[b], gsem[b])

      return 0

    lax.fori_loop(0, nrounds, round_body, 0)
    for b in range(NBUF):
      pltpu.make_async_copy(rows_v.at[b], agg_sh.at[dst_v.at[0]],
                            ssem[b]).wait()

    plsc.subcore_barrier()

    def drain_body(i, _):
      r0 = s * rows_per_tile + i * CHUNK
      pltpu.sync_copy(agg_sh.at[pl.ds(r0, CHUNK)], rows_v.at[0])
      pltpu.sync_copy(rows_v.at[0], out_hbm.at[c, pl.ds(r0, CHUNK)])
      return 0

    lax.fori_loop(0, drain_steps, drain_body, 0)

  return msgpass


def _make_degree():
  rows_per_tile = NPAD // NS
  drain_steps = rows_per_tile // CHUNK
  mesh = plsc.VectorSubcoreMesh(
      core_axis_name="c", subcore_axis_name="s",
      num_cores=NC, num_subcores=NS)

  @functools.partial(
      pl.kernel,
      out_type=jax.ShapeDtypeStruct((NC, NPAD, DEGW), jnp.float32),
      mesh=mesh,
      compiler_params=pltpu.CompilerParams(use_tc_tiling_on_sc=False),
      scratch_types=[
          pltpu.VMEM((NCPT, CHUNK), jnp.int32),
          pltpu.VMEM((CHUNK, DEGW), jnp.float32),
          pltpu.VMEM_SHARED((NPAD, DEGW), jnp.float32),
          [pltpu.SemaphoreType.DMA] * NBUF,
      ],
  )
  def degree(dst_hbm, out_hbm, dst_v, buf_v, deg_sh, ssem):
    c = lax.axis_index("c")
    s = lax.axis_index("s")
    wid = c * NS + s

    _zero_rows(buf_v, CHUNK, DEGW)

    def zbody(i, _):
      pltpu.sync_copy(buf_v, deg_sh.at[pl.ds(s * rows_per_tile + i * CHUNK,
                                             CHUNK)])
      return 0

    lax.fori_loop(0, drain_steps, zbody, 0)

    pltpu.sync_copy(dst_hbm.at[wid], dst_v)
    _fill_rows(buf_v, CHUNK, DEGW, 1.0)

    plsc.subcore_barrier()

    nrounds = NCPT // NBUF
    for b in range(NBUF):
      pltpu.async_copy(buf_v, deg_sh.at[dst_v.at[b]], ssem[b], add=True)

    def round_body(t, _):
      for b in range(NBUF):
        jn = t * NBUF + b + NBUF

        @pl.when(jn < NCPT)
        def _():
          pltpu.make_async_copy(buf_v, deg_sh.at[dst_v.at[0]], ssem[b]).wait()
          pltpu.async_copy(buf_v, deg_sh.at[dst_v.at[jn]], ssem[b], add=True)

      return 0

    lax.fori_loop(0, nrounds, round_body, 0)
    for b in range(NBUF):
      pltpu.make_async_copy(buf_v, deg_sh.at[dst_v.at[0]], ssem[b]).wait()

    plsc.subcore_barrier()

    def drain_body(i, _):
      r0 = s * rows_per_tile + i * CHUNK
      pltpu.sync_copy(deg_sh.at[pl.ds(r0, CHUNK)], buf_v)
      pltpu.sync_copy(buf_v, out_hbm.at[c, pl.ds(r0, CHUNK)])
      return 0

    lax.fori_loop(0, drain_steps, drain_body, 0)

  return degree


def _dot_t(a, w):
  return lax.dot_general(a, w, (((1,), (1,)), ((), ())),
                         preferred_element_type=jnp.float32)


def _tc_first(degp, x, w1):
  def body(degp_ref, x_ref, w1_ref, dis_ref, hs1_ref):
    deg = (degp_ref[0, :N, 0:1] + degp_ref[1, :N, 0:1]) + 1.0
    dis = lax.rsqrt(deg)
    h1 = _dot_t(x_ref[...], w1_ref[...])
    dis_ref[...] = dis
    hs1_ref[...] = h1 * dis

  return pl.pallas_call(
      body,
      out_shape=[
          jax.ShapeDtypeStruct((N, 1), jnp.float32),
          jax.ShapeDtypeStruct((N, H), jnp.float32),
      ],
  )(degp, x, w1)


def _tc_mid(aggp, hs, dis, b, gam, bet, wn, width):
  wn_out = wn.shape[0]

  def body(aggp_ref, hs_ref, dis_ref, b_ref, g_ref, be_ref, wn_ref, o_ref):
    dis = dis_ref[...]
    agg = aggp_ref[0, :N, :] + aggp_ref[1, :N, :] + hs_ref[...]
    out = agg * dis + b_ref[...]
    mu = jnp.mean(out, axis=0, keepdims=True)
    xc = out - mu
    var = jnp.mean(xc * xc, axis=0, keepdims=True)
    y = g_ref[...] * xc * lax.rsqrt(var + EPS) + be_ref[...]
    y = jnp.maximum(y, 0.0)
    o_ref[...] = _dot_t(y, wn_ref[...]) * dis

  return pl.pallas_call(
      body,
      out_shape=jax.ShapeDtypeStruct((N, wn_out), jnp.float32),
  )(aggp, hs, dis, b.reshape(1, width), gam.reshape(1, width),
    bet.reshape(1, width), wn)


def _tc_final(aggp, hs, dis, b, gam, bet, batch2d, wc1, bc1, wc2, bc2):
  def body(aggp_ref, hs_ref, dis_ref, b_ref, g_ref, be_ref, batch_ref,
           wc1_ref, bc1_ref, wc2_ref, bc2_ref, logit_ref, gcat_ref):
    dis = dis_ref[...]
    agg = aggp_ref[0, :N, :] + aggp_ref[1, :N, :] + hs_ref[...]
    out = agg * dis + b_ref[...]
    mu = jnp.mean(out, axis=0, keepdims=True)
    xc = out - mu
    var = jnp.mean(xc * xc, axis=0, keepdims=True)
    h = g_ref[...] * xc * lax.rsqrt(var + EPS) + be_ref[...]
    h = jnp.maximum(h, 0.0)

    batch = batch_ref[...]
    gids = lax.broadcasted_iota(jnp.int32, (1, G), 1)
    onehot = (batch == gids).astype(jnp.float32)
    h_aug = jnp.concatenate([h, jnp.ones((N, 1), jnp.float32)], axis=1)
    sums_aug = lax.dot_general(onehot, h_aug, (((0,), (0,)), ((), ())),
                               preferred_element_type=jnp.float32)
    counts = sums_aug[:, O:O + 1]
    mean_pool = sums_aug[:, :O] / jnp.maximum(counts, 1.0)
    gcat_ref[:, :O] = mean_pool

    def max_body(gi, _):
      mask = batch == gi
      m = jnp.max(jnp.where(mask, h, -jnp.inf), axis=0, keepdims=True)
      gcat_ref[pl.ds(gi, 1), pl.ds(O, O)] = m
      return 0

    lax.fori_loop(0, G, max_body, 0)

    gcat = gcat_ref[...]
    z = jnp.maximum(_dot_t(gcat, wc1_ref[...]) + bc1_ref[...], 0.0)
    logit_ref[...] = jnp.sum(z * wc2_ref[...], axis=1, keepdims=True) + bc2_ref[0, 0]

  return pl.pallas_call(
      body,
      out_shape=[
          jax.ShapeDtypeStruct((G, 1), jnp.float32),
          jax.ShapeDtypeStruct((G, 2 * O), jnp.float32),
      ],
  )(aggp, hs, dis, b.reshape(1, O), gam.reshape(1, O), bet.reshape(1, O),
    batch2d, wc1, bc1.reshape(1, O), wc2, bc2.reshape(1, 1))


_msgpass_h = _make_msgpass(H)
_msgpass_o = _make_msgpass(O)
_degree = _make_degree()


def kernel(x, edge_index, batch,
           W1, b1, g1, be1,
           W2, b2, g2, be2,
           W3, b3, g3, be3,
           Wc1, bc1, Wc2, bc2):
  pad = EPAD - E
  pad_dst = N + jnp.arange(pad, dtype=jnp.int32) % (NPAD - N)
  src = jnp.concatenate([edge_index[0], jnp.zeros((pad,), jnp.int32)])
  dst = jnp.concatenate([edge_index[1], pad_dst])
  src_m = src.reshape(NW, NCPT, CHUNK)
  dst_m = dst.reshape(NW, NCPT, CHUNK)
  batch2d = batch.reshape(N, 1)

  degp = _degree(dst_m)
  dis, hs1 = _tc_first(degp, x, W1)
  agg1 = _msgpass_h(src_m, dst_m, hs1)
  hs2 = _tc_mid(agg1, hs1, dis, b1, g1, be1, W2, H)
  agg2 = _msgpass_h(src_m, dst_m, hs2)
  hs3 = _tc_mid(agg2, hs2, dis, b2, g2, be2, W3, H)
  agg3 = _msgpass_o(src_m, dst_m, hs3)
  logit2d, gcat = _tc_final(agg3, hs3, dis, b3, g3, be3, batch2d,
                            Wc1, bc1, Wc2, bc2)
  return logit2d.reshape(-1), gcat
